# baseline (speedup 1.0000x reference)
import jax
import jax.numpy as jnp
from jax import lax
from jax.experimental import pallas as pl
from jax.experimental.pallas import tpu as pltpu

_B, _K, _D, _N = 128, 32, 256, 8192
_R = 1024
_BPS = _R // _K
_STEPS = (_B * _K) // _R
_NC = 32
_CN = _N // _NC


def _cluster_body(x_ref, c_ref, out_ref, cov_ref, c2_ref, s_ref):
    step = pl.program_id(0)

    @pl.when(step == 0)
    def _init():
        cov_ref[...] = jnp.zeros_like(cov_ref)
        c = c_ref[...]
        c2_ref[...] = lax.dot_general(
            jnp.full((8, _D), 0.5, jnp.float32), c * c,
            (((1,), (1,)), ((), ())),
            preferred_element_type=jnp.float32)

    x = x_ref[...]
    m = None
    for t in range(_NC):
        cols = slice(t * _CN, (t + 1) * _CN)
        sc = lax.dot_general(
            x, c_ref[cols, :], (((1,), (1,)), ((), ())),
            preferred_element_type=jnp.float32)
        sc = sc - c2_ref[0:1, cols]
        s_ref[:, cols] = sc
        mt = jnp.max(sc, axis=1, keepdims=True)
        m = mt if m is None else jnp.maximum(m, mt)

    g = s_ref[...] - m
    for b in range(_BPS):
        gb = jnp.max(g[b * _K:(b + 1) * _K, :], axis=0, keepdims=True)
        cov_ref[b:b + 1, :] += jnp.where(gb >= 0.0, 1.0, 0.0)

    @pl.when(step == _STEPS - 1)
    def _fini():
        coverage = jnp.sum(cov_ref[...], axis=0, keepdims=True)
        prob = coverage / (_B * _K)
        ent = -jnp.sum(prob * jnp.log(prob + 1e-10))
        out_ref[...] = ent[None, None]


def kernel(selected_frames, cluster_centers):
    x = selected_frames.reshape(_B * _K, _D)
    out = pl.pallas_call(
        _cluster_body,
        grid=(_STEPS,),
        in_specs=[
            pl.BlockSpec((_R, _D), lambda i: (i, 0)),
            pl.BlockSpec((_N, _D), lambda i: (0, 0)),
        ],
        out_specs=pl.BlockSpec((1, 1), lambda i: (0, 0)),
        out_shape=jax.ShapeDtypeStruct((1, 1), jnp.float32),
        scratch_shapes=[
            pltpu.VMEM((_BPS, _N), jnp.float32),
            pltpu.VMEM((8, _N), jnp.float32),
            pltpu.VMEM((_R, _N), jnp.float32),
        ],
    )(x, cluster_centers)
    return out[0, 0]
